# baseline (speedup 1.0000x reference)
import functools
import jax
import jax.numpy as jnp
from jax import lax
from jax.experimental import pallas as pl
from jax.experimental.pallas import tpu as pltpu, tpu_sc as plsc

VOCAB = 100000
D = 128
B = 4096
HIST = 50
EPS = 1e-5

NC, NS, LANES = 2, 16, 16
NW = NC * NS
N = B * HIST
ROWS_PER_W = N // NW
G = 128
NCHUNK = ROWS_PER_W // G
NVJ = D // LANES


def _ln_body(x_hbm, table_hbm, gamma_hbm, beta_hbm, out_hbm,
             idx_v, rows0, rows1, gamma_v, beta_v, means_v, ys_v,
             gsem0, gsem1, psem0, psem1):
    wid = lax.axis_index("s") * NC + lax.axis_index("c")
    base = wid * ROWS_PER_W

    pltpu.sync_copy(x_hbm.at[pl.ds(base, ROWS_PER_W)], idx_v)
    pltpu.sync_copy(gamma_hbm, gamma_v)
    pltpu.sync_copy(beta_hbm, beta_v)

    gvs = [gamma_v[pl.ds(LANES * j, LANES)] for j in range(NVJ)]
    bvs = [beta_v[pl.ds(LANES * j, LANES)] for j in range(NVJ)]

    inv_d = jnp.float32(1.0 / D)
    bufs = (rows0, rows1)
    gsems = (gsem0, gsem1)
    psems = (psem0, psem1)

    def start_gather(c, buf, gsem):
        pltpu.async_copy(table_hbm.at[idx_v.at[pl.ds(c * G, G)]], buf, gsem)

    def wait_gather(buf, gsem):
        pltpu.make_async_copy(out_hbm.at[pl.ds(0, G)], buf, gsem).wait()

    def start_put(c, buf, psem):
        pltpu.async_copy(buf, out_hbm.at[pl.ds(base + c * G, G)], psem)

    def wait_put(buf, psem):
        pltpu.make_async_copy(buf, out_hbm.at[pl.ds(0, G)], psem).wait()

    iota = lax.iota(jnp.int32, LANES)
    _bcast_dnums = lax.GatherDimensionNumbers(
        offset_dims=(), collapsed_slice_dims=(0,), start_index_map=(0,))

    def _lane_bcast(vec, lane):
        return lax.gather(
            vec, lane[:, None], _bcast_dnums, (1,),
            mode=lax.GatherScatterMode.PROMISE_IN_BOUNDS)

    def compute_chunk(buf, means_v, ys_v):
        def group_body(t, carry):
            r0 = t * LANES
            row_ids = r0 + iota
            nacc = 4
            ss = [None] * nacc
            qq = [None] * nacc
            for j in range(D):
                col = plsc.load_gather(
                    buf, [row_ids, jnp.full((LANES,), j, jnp.int32)])
                a = j % nacc
                if ss[a] is None:
                    ss[a] = col
                    qq[a] = col * col
                else:
                    ss[a] = ss[a] + col
                    qq[a] = qq[a] + col * col
            s = (ss[0] + ss[1]) + (ss[2] + ss[3])
            sq = (qq[0] + qq[1]) + (qq[2] + qq[3])
            mean = s * inv_d
            var = sq * inv_d - mean * mean
            vv = var + jnp.float32(EPS)
            iv = plsc.bitcast(vv, jnp.int32)
            y = plsc.bitcast(jnp.int32(0x5F3759DF) - (iv >> 1), jnp.float32)
            half = jnp.float32(0.5) * vv
            for _ in range(3):
                y = y * (jnp.float32(1.5) - half * y * y)
            for i in range(LANES):
                lane = jnp.full((LANES,), i, jnp.int32)
                m_b = _lane_bcast(mean, lane)
                y_b = _lane_bcast(y, lane)
                for j in range(NVJ):
                    v = buf[r0 + i, pl.ds(LANES * j, LANES)]
                    buf[r0 + i, pl.ds(LANES * j, LANES)] = (
                        (v - m_b) * y_b * gvs[j] + bvs[j])
            return carry

        lax.fori_loop(0, G // LANES, group_body, 0)

    start_gather(0, bufs[0], gsems[0])

    def chunk_body(c, carry):
        for par in (0, 1):
            @pl.when(c % 2 == par)
            def _():
                buf, gsem, psem = bufs[par], gsems[par], psems[par]
                obuf, ogsem, opsem = bufs[1 - par], gsems[1 - par], psems[1 - par]

                @pl.when(c + 1 < NCHUNK)
                def _():
                    @pl.when(c >= 1)
                    def _():
                        wait_put(obuf, opsem)
                    start_gather(c + 1, obuf, ogsem)

                wait_gather(buf, gsem)
                compute_chunk(buf, means_v, ys_v)
                start_put(c, buf, psem)
        return carry

    lax.fori_loop(0, NCHUNK, chunk_body, 0)
    wait_put(bufs[(NCHUNK - 1) % 2], psems[(NCHUNK - 1) % 2])
    wait_put(bufs[NCHUNK % 2], psems[NCHUNK % 2])


@jax.jit
def _run(x_flat, table, gamma, beta):
    mesh = plsc.VectorSubcoreMesh(core_axis_name="c", subcore_axis_name="s")
    f = pl.kernel(
        _ln_body,
        out_type=jax.ShapeDtypeStruct((N, D), jnp.float32),
        mesh=mesh,
        scratch_types=[
            pltpu.VMEM((ROWS_PER_W,), jnp.int32),
            pltpu.VMEM((G, D), jnp.float32),
            pltpu.VMEM((G, D), jnp.float32),
            pltpu.VMEM((D,), jnp.float32),
            pltpu.VMEM((D,), jnp.float32),
            pltpu.VMEM((LANES,), jnp.float32),
            pltpu.VMEM((LANES,), jnp.float32),
            pltpu.SemaphoreType.DMA,
            pltpu.SemaphoreType.DMA,
            pltpu.SemaphoreType.DMA,
            pltpu.SemaphoreType.DMA,
        ],
        compiler_params=pltpu.CompilerParams(needs_layout_passes=False),
    )
    return f(x_flat, table, gamma, beta)


def kernel(x, table, gamma, beta):
    x_flat = x.reshape(-1).astype(jnp.int32)
    out = _run(x_flat, table, gamma, beta)
    return out.reshape(B, HIST, D)


# speedup vs baseline: 1.5636x; 1.5636x over previous
import functools
import jax
import jax.numpy as jnp
from jax import lax
from jax.experimental import pallas as pl
from jax.experimental.pallas import tpu as pltpu, tpu_sc as plsc

VOCAB = 100000
D = 128
B = 4096
HIST = 50
EPS = 1e-5

NC, NS, LANES = 2, 16, 16
NW = NC * NS
N = B * HIST
ROWS_PER_W = N // NW
G = 128
NCHUNK = ROWS_PER_W // G
NVJ = D // LANES


def _ln_body(x_hbm, table_hbm, gamma_hbm, beta_hbm, out_hbm,
             idx_v, rows0, rows1, gamma_v, beta_v,
             gsem0, gsem1, psem0, psem1):
    wid = lax.axis_index("s") * NC + lax.axis_index("c")
    base = wid * ROWS_PER_W

    pltpu.sync_copy(x_hbm.at[pl.ds(base, ROWS_PER_W)], idx_v)
    pltpu.sync_copy(gamma_hbm, gamma_v)
    pltpu.sync_copy(beta_hbm, beta_v)

    gvs = [gamma_v[pl.ds(LANES * j, LANES)] for j in range(NVJ)]
    bvs = [beta_v[pl.ds(LANES * j, LANES)] for j in range(NVJ)]

    inv_d = jnp.float32(1.0 / D)
    bufs = (rows0, rows1)
    gsems = (gsem0, gsem1)
    psems = (psem0, psem1)

    def start_gather(c, buf, gsem):
        pltpu.async_copy(table_hbm.at[idx_v.at[pl.ds(c * G, G)]], buf, gsem)

    def wait_gather(buf, gsem):
        pltpu.make_async_copy(out_hbm.at[pl.ds(0, G)], buf, gsem).wait()

    def start_put(c, buf, psem):
        pltpu.async_copy(buf, out_hbm.at[pl.ds(base + c * G, G)], psem)

    def wait_put(buf, psem):
        pltpu.make_async_copy(buf, out_hbm.at[pl.ds(0, G)], psem).wait()

    iota = lax.iota(jnp.int32, LANES)
    _perm_dnums = lax.GatherDimensionNumbers(
        offset_dims=(), collapsed_slice_dims=(0,), start_index_map=(0,))

    def _perm(vec, idx):
        return lax.gather(
            vec, idx[:, None], _perm_dnums, (1,),
            mode=lax.GatherScatterMode.PROMISE_IN_BOUNDS)

    perms = [iota ^ k for k in (1, 2, 4, 8)]

    def compute_chunk(buf):
        def row_body(r, carry):
            vs = [buf[r, pl.ds(LANES * j, LANES)] for j in range(NVJ)]
            s = vs[0]
            sq = vs[0] * vs[0]
            for j in range(1, NVJ):
                s = s + vs[j]
                sq = sq + vs[j] * vs[j]
            for ix in perms:
                s = s + _perm(s, ix)
                sq = sq + _perm(sq, ix)
            mean = s * inv_d
            var = sq * inv_d - mean * mean
            vv = var + jnp.float32(EPS)
            iv = plsc.bitcast(vv, jnp.int32)
            y = plsc.bitcast(jnp.int32(0x5F3759DF) - (iv >> 1), jnp.float32)
            half = jnp.float32(0.5) * vv
            for _ in range(3):
                y = y * (jnp.float32(1.5) - half * y * y)
            for j in range(NVJ):
                buf[r, pl.ds(LANES * j, LANES)] = (
                    (vs[j] - mean) * y * gvs[j] + bvs[j])
            return carry

        lax.fori_loop(0, G, row_body, 0, unroll=2)

    start_gather(0, bufs[0], gsems[0])

    def chunk_body(c, carry):
        for par in (0, 1):
            @pl.when(c % 2 == par)
            def _():
                buf, gsem, psem = bufs[par], gsems[par], psems[par]
                obuf, ogsem, opsem = bufs[1 - par], gsems[1 - par], psems[1 - par]

                @pl.when(c + 1 < NCHUNK)
                def _():
                    @pl.when(c >= 1)
                    def _():
                        wait_put(obuf, opsem)
                    start_gather(c + 1, obuf, ogsem)

                wait_gather(buf, gsem)
                compute_chunk(buf)
                start_put(c, buf, psem)
        return carry

    lax.fori_loop(0, NCHUNK, chunk_body, 0)
    wait_put(bufs[(NCHUNK - 1) % 2], psems[(NCHUNK - 1) % 2])
    wait_put(bufs[NCHUNK % 2], psems[NCHUNK % 2])


@jax.jit
def _run(x_flat, table, gamma, beta):
    mesh = plsc.VectorSubcoreMesh(core_axis_name="c", subcore_axis_name="s")
    f = pl.kernel(
        _ln_body,
        out_type=jax.ShapeDtypeStruct((N, D), jnp.float32),
        mesh=mesh,
        scratch_types=[
            pltpu.VMEM((ROWS_PER_W,), jnp.int32),
            pltpu.VMEM((G, D), jnp.float32),
            pltpu.VMEM((G, D), jnp.float32),
            pltpu.VMEM((D,), jnp.float32),
            pltpu.VMEM((D,), jnp.float32),
            pltpu.SemaphoreType.DMA,
            pltpu.SemaphoreType.DMA,
            pltpu.SemaphoreType.DMA,
            pltpu.SemaphoreType.DMA,
        ],
        compiler_params=pltpu.CompilerParams(needs_layout_passes=False),
    )
    return f(x_flat, table, gamma, beta)


def kernel(x, table, gamma, beta):
    x_flat = x.reshape(-1).astype(jnp.int32)
    out = _run(x_flat, table, gamma, beta)
    return out.reshape(B, HIST, D)
